# baseline (speedup 1.0000x reference)
import functools

import jax
import jax.numpy as jnp
from jax import lax
from jax.experimental import pallas as pl
from jax.experimental.pallas import tpu as pltpu
from jax.experimental.pallas import tpu_sc as plsc

NUM_CODES = 8192
DIM = 256
NUM_TOKENS = 8192

TOK_TILE = 1024
CODE_TILE = 1024


def _argmin_body(z_ref, e_ref, idx_ref, min_ref, loss_ref):
    i = pl.program_id(0)
    j = pl.program_id(1)
    nj = pl.num_programs(1)

    zt = z_ref[...]
    et = e_ref[...]
    sz = jnp.sum(zt ** 2, axis=1, keepdims=True)
    se = jnp.sum(et ** 2, axis=1)
    mm = lax.dot_general(zt, et, (((1,), (1,)), ((), ())),
                         preferred_element_type=jnp.float32)
    d = (sz - 2.0 * mm) + se[None, :]

    m = jnp.min(d, axis=1, keepdims=True)
    cols = lax.broadcasted_iota(jnp.int32, d.shape, 1) + j * CODE_TILE
    cand = jnp.where(d == m, cols, jnp.int32(2**31 - 1))
    idxt = jnp.min(cand, axis=1)
    mt = m[:, 0]

    @pl.when(j == 0)
    def _():
        idx_ref[...] = idxt
        min_ref[...] = mt

    @pl.when(j > 0)
    def _():
        better = mt < min_ref[...]
        idx_ref[...] = jnp.where(better, idxt, idx_ref[...])
        min_ref[...] = jnp.where(better, mt, min_ref[...])

    @pl.when(j == nj - 1)
    def _():
        tile_loss = jnp.sum(min_ref[...])

        @pl.when(i == 0)
        def _():
            loss_ref[0, 0] = tile_loss

        @pl.when(i > 0)
        def _():
            loss_ref[0, 0] = loss_ref[0, 0] + tile_loss


def _argmin_search(z_flat, emb_weight):
    n_i = NUM_TOKENS // TOK_TILE
    n_j = NUM_CODES // CODE_TILE
    return pl.pallas_call(
        _argmin_body,
        grid=(n_i, n_j),
        in_specs=[
            pl.BlockSpec((TOK_TILE, DIM), lambda i, j: (i, 0)),
            pl.BlockSpec((CODE_TILE, DIM), lambda i, j: (j, 0)),
        ],
        out_specs=[
            pl.BlockSpec((TOK_TILE,), lambda i, j: (i,)),
            pl.BlockSpec((TOK_TILE,), lambda i, j: (i,)),
            pl.BlockSpec((1, 1), lambda i, j: (0, 0),
                         memory_space=pltpu.SMEM),
        ],
        out_shape=[
            jax.ShapeDtypeStruct((NUM_TOKENS,), jnp.int32),
            jax.ShapeDtypeStruct((NUM_TOKENS,), jnp.float32),
            jax.ShapeDtypeStruct((1, 1), jnp.float32),
        ],
        compiler_params=pltpu.CompilerParams(
            dimension_semantics=("arbitrary", "arbitrary"),
        ),
    )(z_flat, emb_weight)


def _make_sc_gather():
    info = plsc.get_sparse_core_info()
    nw = info.num_cores * info.num_subcores
    b_per_w = NUM_TOKENS // nw

    mesh = plsc.VectorSubcoreMesh(core_axis_name="c", subcore_axis_name="s")

    @functools.partial(
        pl.kernel,
        mesh=mesh,
        out_type=jax.ShapeDtypeStruct((NUM_TOKENS, DIM), jnp.float32),
        scratch_types=[
            pltpu.VMEM((b_per_w,), jnp.int32),
            pltpu.VMEM((b_per_w, DIM), jnp.float32),
            pltpu.SemaphoreType.DMA,
        ],
    )
    def gather_kernel(emb_hbm, idx_hbm, out_hbm, idx_v, rows_v, sem):
        wid = lax.axis_index("s") * info.num_cores + lax.axis_index("c")
        base = wid * b_per_w
        pltpu.sync_copy(idx_hbm.at[pl.ds(base, b_per_w)], idx_v)
        pltpu.async_copy(emb_hbm.at[idx_v], rows_v, sem).wait()
        pltpu.sync_copy(rows_v, out_hbm.at[pl.ds(base, b_per_w)])

    return gather_kernel


def kernel(z, emb_weight):
    B, D, H, W = z.shape
    z_flat = jnp.transpose(z, (0, 2, 3, 1)).reshape(-1, D)

    indices, _dmin, loss_sum = _argmin_search(z_flat, emb_weight)

    z_q_flat = _make_sc_gather()(emb_weight, indices)

    z_q = z_q_flat.reshape(B, H, W, D)
    z_q = jnp.transpose(z_q, (0, 3, 1, 2))
    commitment_loss = (loss_sum[0, 0] / jnp.float32(B * D * H * W)).reshape(())
    z_q_st = z + lax.stop_gradient(z_q - z)
    indices_grid = indices.reshape(B, H, W)
    return (z_q_st, commitment_loss, indices_grid)


# speedup vs baseline: 1.6388x; 1.6388x over previous
import functools

import jax
import jax.numpy as jnp
from jax import lax
from jax.experimental import pallas as pl
from jax.experimental.pallas import tpu as pltpu
from jax.experimental.pallas import tpu_sc as plsc

NUM_CODES = 8192
DIM = 256
NUM_TOKENS = 8192

TOK_TILE = 1024
CODE_TILE = 8192


def _argmin_body(z_ref, e2_ref, idx_ref, min_ref, loss_ref):
    i = pl.program_id(0)
    j = pl.program_id(1)
    nj = pl.num_programs(1)

    zt = z_ref[...]
    e2t = e2_ref[...]
    sz = jnp.sum(zt ** 2, axis=1, keepdims=True)
    se = 0.25 * jnp.sum(e2t ** 2, axis=1)
    mm2 = lax.dot_general(zt, e2t, (((1,), (1,)), ((), ())),
                          preferred_element_type=jnp.float32)
    d = (sz + mm2) + se[None, :]

    LB = 128
    nb = CODE_TILE // LB
    m = d[:, 0:LB]
    for a in range(1, nb):
        m = jnp.minimum(m, d[:, a * LB:(a + 1) * LB])
    mrow = jnp.min(m, axis=1, keepdims=True)

    lane_iota = lax.broadcasted_iota(jnp.int32, (TOK_TILE, LB), 1)
    big = jnp.int32(2**31 - 1)
    cand = None
    for a in range(nb):
        ca = jnp.where(d[:, a * LB:(a + 1) * LB] == mrow,
                       lane_iota + (j * CODE_TILE + a * LB), big)
        cand = ca if cand is None else jnp.minimum(cand, ca)
    idxt = jnp.min(cand, axis=1)
    mt = mrow[:, 0]

    @pl.when(j == 0)
    def _():
        idx_ref[...] = idxt
        min_ref[...] = mt

    @pl.when(j > 0)
    def _():
        better = mt < min_ref[...]
        idx_ref[...] = jnp.where(better, idxt, idx_ref[...])
        min_ref[...] = jnp.where(better, mt, min_ref[...])

    @pl.when(j == nj - 1)
    def _():
        tile_loss = jnp.sum(min_ref[...])

        @pl.when(i == 0)
        def _():
            loss_ref[0, 0] = tile_loss

        @pl.when(i > 0)
        def _():
            loss_ref[0, 0] = loss_ref[0, 0] + tile_loss


def _argmin_search(z_flat, emb2):
    n_i = NUM_TOKENS // TOK_TILE
    n_j = NUM_CODES // CODE_TILE
    return pl.pallas_call(
        _argmin_body,
        grid=(n_i, n_j),
        in_specs=[
            pl.BlockSpec((TOK_TILE, DIM), lambda i, j: (i, 0)),
            pl.BlockSpec((CODE_TILE, DIM), lambda i, j: (j, 0)),
        ],
        out_specs=[
            pl.BlockSpec((TOK_TILE,), lambda i, j: (i,)),
            pl.BlockSpec((TOK_TILE,), lambda i, j: (i,)),
            pl.BlockSpec((1, 1), lambda i, j: (0, 0),
                         memory_space=pltpu.SMEM),
        ],
        out_shape=[
            jax.ShapeDtypeStruct((NUM_TOKENS,), jnp.int32),
            jax.ShapeDtypeStruct((NUM_TOKENS,), jnp.float32),
            jax.ShapeDtypeStruct((1, 1), jnp.float32),
        ],
        compiler_params=pltpu.CompilerParams(
            dimension_semantics=("arbitrary", "arbitrary"),
        ),
    )(z_flat, emb2)


def _make_sc_gather():
    info = plsc.get_sparse_core_info()
    nw = info.num_cores * info.num_subcores
    b_per_w = NUM_TOKENS // nw

    mesh = plsc.VectorSubcoreMesh(core_axis_name="c", subcore_axis_name="s")

    @functools.partial(
        pl.kernel,
        mesh=mesh,
        out_type=jax.ShapeDtypeStruct((NUM_TOKENS, DIM), jnp.float32),
        scratch_types=[
            pltpu.VMEM((b_per_w,), jnp.int32),
            pltpu.VMEM((b_per_w, DIM), jnp.float32),
            pltpu.SemaphoreType.DMA,
        ],
    )
    def gather_kernel(emb_hbm, idx_hbm, out_hbm, idx_v, rows_v, sem):
        wid = lax.axis_index("s") * info.num_cores + lax.axis_index("c")
        base = wid * b_per_w
        pltpu.sync_copy(idx_hbm.at[pl.ds(base, b_per_w)], idx_v)
        pltpu.async_copy(emb_hbm.at[idx_v], rows_v, sem).wait()
        pltpu.sync_copy(rows_v, out_hbm.at[pl.ds(base, b_per_w)])

    return gather_kernel


def kernel(z, emb_weight):
    B, D, H, W = z.shape
    z_flat = jnp.transpose(z, (0, 2, 3, 1)).reshape(-1, D)

    emb2 = -2.0 * emb_weight
    indices, _dmin, loss_sum = _argmin_search(z_flat, emb2)

    z_q_flat = _make_sc_gather()(emb_weight, indices)

    z_q = z_q_flat.reshape(B, H, W, D)
    z_q = jnp.transpose(z_q, (0, 3, 1, 2))
    commitment_loss = (loss_sum[0, 0] / jnp.float32(B * D * H * W)).reshape(())
    z_q_st = z + lax.stop_gradient(z_q - z)
    indices_grid = indices.reshape(B, H, W)
    return (z_q_st, commitment_loss, indices_grid)
